# baseline (speedup 1.0000x reference)
import functools

import jax
import jax.numpy as jnp
from jax.experimental import pallas as pl
from jax.experimental.pallas import tpu as pltpu

E = 8
K = 2
T = 2048
HID = 1024
EXP = 2048
TM = 128
R = T * K
NT = R // TM + E
RPAD = NT * TM


def _router_body(x_ref, wg_ref, logits_ref):
    logits_ref[...] = jax.lax.dot_general(
        x_ref[...], wg_ref[...], (((1,), (1,)), ((), ())),
        preferred_element_type=jnp.float32)


def _moe_body(g_ref, n_ref, xs_ref, w1_ref, w3_ref, w2_ref, ws_ref, ys_ref):
    i = pl.program_id(0)

    @pl.when(i < n_ref[0])
    def _():
        x = xs_ref[...]
        a = jax.lax.dot_general(x, w1_ref[0], (((1,), (1,)), ((), ())),
                                preferred_element_type=jnp.float32)
        b = jax.lax.dot_general(x, w3_ref[0], (((1,), (1,)), ((), ())),
                                preferred_element_type=jnp.float32)
        h = (a * jax.nn.sigmoid(a)) * b
        y = jax.lax.dot_general(h, w2_ref[0], (((1,), (1,)), ((), ())),
                                preferred_element_type=jnp.float32)
        ys_ref[...] = y * ws_ref[...]


def kernel(x, Wg, W1, W2, W3, layer_idx):
    del layer_idx

    logits = pl.pallas_call(
        _router_body,
        grid=(T // TM,),
        in_specs=[
            pl.BlockSpec((TM, HID), lambda i: (i, 0)),
            pl.BlockSpec((E, HID), lambda i: (0, 0)),
        ],
        out_specs=pl.BlockSpec((TM, E), lambda i: (i, 0)),
        out_shape=jax.ShapeDtypeStruct((T, E), jnp.float32),
    )(x, Wg)

    topv, topi = jax.lax.top_k(logits, K)
    weights = jax.nn.softmax(topv, axis=-1)
    probs = jax.nn.softmax(logits, axis=-1)
    usage = probs.mean(axis=0)
    lb_loss = E * jnp.sum(usage * usage)

    flat_e = topi.reshape(-1).astype(jnp.int32)
    oh = (flat_e[:, None] == jnp.arange(E, dtype=jnp.int32)[None, :])
    oh = oh.astype(jnp.int32)
    within = jnp.cumsum(oh, axis=0) - oh
    pos = jnp.take_along_axis(within, flat_e[:, None], axis=1)[:, 0]
    counts = oh.sum(axis=0)
    padded = ((counts + TM - 1) // TM) * TM
    ends = jnp.cumsum(padded)
    starts = ends - padded
    dest = starts[flat_e] + pos
    ntiles = ends[-1] // TM

    tok_and_w = jnp.stack(
        [jnp.arange(R, dtype=jnp.int32) // K,
         jax.lax.bitcast_convert_type(weights.reshape(-1), jnp.int32)], axis=1)
    sorted_tw = jnp.zeros((RPAD, 2), jnp.int32).at[dest].set(
        tok_and_w, unique_indices=True, mode="promise_in_bounds")
    sorted_tok = sorted_tw[:, 0]
    ws_sorted = jax.lax.bitcast_convert_type(sorted_tw[:, 1], jnp.float32)
    tile_ends = ends // TM
    g = jnp.sum(jnp.arange(NT, dtype=jnp.int32)[:, None]
                >= tile_ends[None, :], axis=1)
    g = jnp.minimum(g, E - 1).astype(jnp.int32)
    nact = ntiles.reshape(1).astype(jnp.int32)

    xs = jnp.take(x, sorted_tok, axis=0)

    grid_spec = pltpu.PrefetchScalarGridSpec(
        num_scalar_prefetch=2,
        grid=(NT,),
        in_specs=[
            pl.BlockSpec((TM, HID), lambda i, g_r, n_r: (i, 0)),
            pl.BlockSpec((1, EXP, HID), lambda i, g_r, n_r: (g_r[i], 0, 0)),
            pl.BlockSpec((1, EXP, HID), lambda i, g_r, n_r: (g_r[i], 0, 0)),
            pl.BlockSpec((1, HID, EXP), lambda i, g_r, n_r: (g_r[i], 0, 0)),
            pl.BlockSpec((TM, 1), lambda i, g_r, n_r: (i, 0)),
        ],
        out_specs=pl.BlockSpec((TM, HID), lambda i, g_r, n_r: (i, 0)),
    )
    ys = pl.pallas_call(
        _moe_body,
        grid_spec=grid_spec,
        out_shape=jax.ShapeDtypeStruct((RPAD, HID), jnp.float32),
    )(g, nact, xs, W1, W3, W2, ws_sorted[:, None])

    dest_tk = dest.reshape(T, K)
    out = jnp.take(ys, dest_tk[:, 0], axis=0) + jnp.take(ys, dest_tk[:, 1], axis=0)
    return (out, lb_loss)


# speedup vs baseline: 1.3681x; 1.3681x over previous
import functools

import jax
import jax.numpy as jnp
from jax.experimental import pallas as pl
from jax.experimental.pallas import tpu as pltpu

E = 8
K = 2
T = 2048
HID = 1024
EXP = 2048
TM = 256
R = T * K
NT = R // TM + E
RPAD = NT * TM


def _cumsum(x, axis):
    n = x.shape[axis]
    zshape = list(x.shape)
    s = 1
    while s < n:
        zshape[axis] = s
        zeros = jnp.zeros(zshape, x.dtype)
        kept = jax.lax.slice_in_dim(x, 0, n - s, axis=axis)
        x = x + jnp.concatenate([zeros, kept], axis=axis)
        s *= 2
    return x


def _route_body(x_ref, wg_ref, dest_ref, w_ref, counts_ref, lb_ref):
    lg = jax.lax.dot_general(
        wg_ref[...], x_ref[...], (((1,), (1,)), ((), ())),
        preferred_element_type=jnp.float32)
    iota_e = jax.lax.broadcasted_iota(jnp.int32, (E, T), 0)

    m1 = jnp.max(lg, axis=0, keepdims=True)
    idx1 = jnp.min(jnp.where(lg >= m1, iota_e, E), axis=0, keepdims=True)
    oh1 = iota_e == idx1
    masked = jnp.where(oh1, -jnp.inf, lg)
    m2 = jnp.max(masked, axis=0, keepdims=True)
    idx2 = jnp.min(jnp.where(masked >= m2, iota_e, E), axis=0, keepdims=True)
    oh2 = iota_e == idx2

    e2 = jnp.exp(m2 - m1)
    w0 = 1.0 / (1.0 + e2)
    w1 = e2 / (1.0 + e2)
    w_ref[...] = jnp.concatenate([w0, w1], axis=0)

    p = jnp.exp(lg - m1)
    probs = p / jnp.sum(p, axis=0, keepdims=True)
    usage = jnp.sum(probs, axis=1) * (1.0 / T)
    lb_ref[0, 0] = E * jnp.sum(usage * usage)

    a1 = oh1.astype(jnp.int32)
    a2 = oh2.astype(jnp.int32)
    c1 = _cumsum(a1, 1) - a1
    n1 = jnp.sum(a1, axis=1, keepdims=True)
    c2 = n1 + _cumsum(a2, 1) - a2
    counts = n1 + jnp.sum(a2, axis=1, keepdims=True)
    padded = ((counts + TM - 1) // TM) * TM
    starts = _cumsum(padded, 0) - padded
    d0 = jnp.sum(jnp.where(oh1, starts + c1, 0), axis=0, keepdims=True)
    d1 = jnp.sum(jnp.where(oh2, starts + c2, 0), axis=0, keepdims=True)
    dest_ref[...] = jnp.concatenate([d0, d1], axis=0)
    counts_ref[...] = counts


def _moe_body(g_ref, n_ref, xs_ref, w1_ref, w3_ref, w2_ref, ws_ref, ys_ref):
    i = pl.program_id(0)

    @pl.when(i < n_ref[0])
    def _():
        x = xs_ref[...]
        a = jax.lax.dot_general(x, w1_ref[0], (((1,), (1,)), ((), ())),
                                preferred_element_type=jnp.float32)
        b = jax.lax.dot_general(x, w3_ref[0], (((1,), (1,)), ((), ())),
                                preferred_element_type=jnp.float32)
        h = (a * jax.nn.sigmoid(a)) * b
        y = jax.lax.dot_general(h, w2_ref[0], (((1,), (1,)), ((), ())),
                                preferred_element_type=jnp.float32)
        ys_ref[...] = y * ws_ref[...]


def kernel(x, Wg, W1, W2, W3, layer_idx):
    del layer_idx

    dest01, w01, counts, lb = pl.pallas_call(
        _route_body,
        in_specs=[
            pl.BlockSpec((T, HID), lambda: (0, 0)),
            pl.BlockSpec((E, HID), lambda: (0, 0)),
        ],
        out_specs=[
            pl.BlockSpec((K, T), lambda: (0, 0)),
            pl.BlockSpec((K, T), lambda: (0, 0)),
            pl.BlockSpec((E, 1), lambda: (0, 0)),
            pl.BlockSpec(memory_space=pltpu.SMEM),
        ],
        out_shape=[
            jax.ShapeDtypeStruct((K, T), jnp.int32),
            jax.ShapeDtypeStruct((K, T), jnp.float32),
            jax.ShapeDtypeStruct((E, 1), jnp.int32),
            jax.ShapeDtypeStruct((1, 1), jnp.float32),
        ],
    )(x, Wg)
    lb_loss = lb[0, 0]

    counts = counts[:, 0]
    padded = ((counts + TM - 1) // TM) * TM
    ends = jnp.cumsum(padded)
    ntiles = ends[-1] // TM
    dest = dest01.T.reshape(-1)
    weights = w01.T.reshape(-1)

    tok_and_w = jnp.stack(
        [jnp.arange(R, dtype=jnp.int32) // K,
         jax.lax.bitcast_convert_type(weights, jnp.int32)], axis=1)
    sorted_tw = jnp.zeros((RPAD, 2), jnp.int32).at[dest].set(
        tok_and_w, unique_indices=True, mode="promise_in_bounds")
    sorted_tok = sorted_tw[:, 0]
    ws_sorted = jax.lax.bitcast_convert_type(sorted_tw[:, 1], jnp.float32)
    tile_ends = ends // TM
    g = jnp.sum(jnp.arange(NT, dtype=jnp.int32)[:, None]
                >= tile_ends[None, :], axis=1)
    g = jnp.minimum(g, E - 1).astype(jnp.int32)
    nact = ntiles.reshape(1).astype(jnp.int32)

    xs = jnp.take(x, sorted_tok, axis=0)

    grid_spec = pltpu.PrefetchScalarGridSpec(
        num_scalar_prefetch=2,
        grid=(NT,),
        in_specs=[
            pl.BlockSpec((TM, HID), lambda i, g_r, n_r: (i, 0)),
            pl.BlockSpec((1, EXP, HID), lambda i, g_r, n_r: (g_r[i], 0, 0)),
            pl.BlockSpec((1, EXP, HID), lambda i, g_r, n_r: (g_r[i], 0, 0)),
            pl.BlockSpec((1, HID, EXP), lambda i, g_r, n_r: (g_r[i], 0, 0)),
            pl.BlockSpec((TM, 1), lambda i, g_r, n_r: (i, 0)),
        ],
        out_specs=pl.BlockSpec((TM, HID), lambda i, g_r, n_r: (i, 0)),
    )
    ys = pl.pallas_call(
        _moe_body,
        grid_spec=grid_spec,
        out_shape=jax.ShapeDtypeStruct((RPAD, HID), jnp.float32),
    )(g, nact, xs, W1, W3, W2, ws_sorted[:, None])

    dest_tk = dest.reshape(T, K)
    out = jnp.take(ys, dest_tk[:, 0], axis=0) + jnp.take(ys, dest_tk[:, 1], axis=0)
    return (out, lb_loss)
